# baseline (speedup 1.0000x reference)
import functools

import jax
import jax.numpy as jnp
import numpy as np
from jax import lax
from jax.experimental import pallas as pl
from jax.experimental.pallas import tpu as pltpu
from jax.experimental.pallas import tpu_sc as plsc

N = 10000
E = 320000
F = 128
H = 128
R = 3

NC = 2
NS = 16
NW = NC * NS

EPW = E // NW
CH = 80
NCHUNK = EPW // CH
EPS = E // NS
NCHUNK_CNT = EPS // CH
NRP = 30720
RPT = 624

BN = 1000
NB = N // BN


def _mm_body(x_ref, wrel_ref, wroot_ref, b_ref, o_ref, oroot_ref):
    xb = x_ref[...]
    for r in range(R):
        o_ref[r] = jnp.dot(xb, wrel_ref[r], preferred_element_type=jnp.float32)
    oroot_ref[...] = (
        jnp.dot(xb, wroot_ref[...], preferred_element_type=jnp.float32)
        + b_ref[...])


def _matmul_rel(x, wrel, wroot, b):
    return pl.pallas_call(
        _mm_body,
        grid=(NB,),
        in_specs=[
            pl.BlockSpec((BN, F), lambda nb: (nb, 0)),
            pl.BlockSpec((R, F, H), lambda nb: (0, 0, 0)),
            pl.BlockSpec((F, H), lambda nb: (0, 0)),
            pl.BlockSpec((1, H), lambda nb: (0, 0)),
        ],
        out_specs=[
            pl.BlockSpec((R, BN, H), lambda nb: (0, nb, 0)),
            pl.BlockSpec((BN, H), lambda nb: (nb, 0)),
        ],
        out_shape=[
            jax.ShapeDtypeStruct((R, N, H), jnp.float32),
            jax.ShapeDtypeStruct((N, H), jnp.float32),
        ],
    )(x, wrel, wroot, b)


def _mm2_body(acc_ref, xroot_ref, wrel_ref, wroot_ref, b_ref, o_ref,
              oroot_ref):
    h = jnp.maximum(acc_ref[0] + acc_ref[1] + xroot_ref[...], 0.0)
    for r in range(R):
        o_ref[r] = jnp.dot(h, wrel_ref[r], preferred_element_type=jnp.float32)
    oroot_ref[...] = (
        jnp.dot(h, wroot_ref[...], preferred_element_type=jnp.float32)
        + b_ref[...])


def _matmul_rel_fused(accp, xroot, wrel, wroot, b):
    return pl.pallas_call(
        _mm2_body,
        grid=(NB,),
        in_specs=[
            pl.BlockSpec((2, BN, H), lambda nb: (0, nb, 0)),
            pl.BlockSpec((BN, H), lambda nb: (nb, 0)),
            pl.BlockSpec((R, H, H), lambda nb: (0, 0, 0)),
            pl.BlockSpec((H, H), lambda nb: (0, 0)),
            pl.BlockSpec((1, H), lambda nb: (0, 0)),
        ],
        out_specs=[
            pl.BlockSpec((R, BN, H), lambda nb: (0, nb, 0)),
            pl.BlockSpec((BN, H), lambda nb: (nb, 0)),
        ],
        out_shape=[
            jax.ShapeDtypeStruct((R, N, H), jnp.float32),
            jax.ShapeDtypeStruct((N, H), jnp.float32),
        ],
    )(accp, xroot, wrel, wroot, b)


def _head_body(acc_ref, xroot_ref, fcw_ref, fcb_ref, o_ref):
    h = jnp.maximum(acc_ref[0] + acc_ref[1] + xroot_ref[...], 0.0)
    z = jnp.sum(h * fcw_ref[...], axis=1, keepdims=True) + fcb_ref[...]
    o_ref[...] = 1.0 / (1.0 + jnp.exp(-z))


def _head(accp, xroot, fcw_row, fcb):
    return pl.pallas_call(
        _head_body,
        grid=(NB,),
        in_specs=[
            pl.BlockSpec((2, BN, H), lambda nb: (0, nb, 0)),
            pl.BlockSpec((BN, H), lambda nb: (nb, 0)),
            pl.BlockSpec((1, H), lambda nb: (0, 0)),
            pl.BlockSpec((1, 1), lambda nb: (0, 0)),
        ],
        out_specs=pl.BlockSpec((BN, 1), lambda nb: (nb, 0)),
        out_shape=jax.ShapeDtypeStruct((N, 1), jnp.float32),
    )(accp, xroot, fcw_row, fcb)


def _scale_body(ei_hbm, et_hbm, scale_hbm, packed_hbm,
                cnt_sh, cntloc, seg2d, et_all, src_half, scb, pcb, onesb, zb,
                sem_in, sem_cnt, sem_src):
    cid = lax.axis_index("c")
    sid = lax.axis_index("s")

    cp1 = pltpu.async_copy(ei_hbm.at[pl.ds(E + sid * EPS, EPS)], et_all,
                           sem_in)
    cp3 = pltpu.async_copy(ei_hbm.at[pl.ds(sid * EPS + cid * EPW, EPW)],
                           src_half, sem_src)

    @pl.loop(0, NRP // NS // 16)
    def _z(i):
        zb[pl.ds(i * 16, 16)] = jnp.zeros((16,), jnp.float32)

    pltpu.sync_copy(zb, cnt_sh.at[pl.ds(sid * (NRP // NS), NRP // NS)])

    @pl.loop(0, CH // 16)
    def _o(j):
        onesb[pl.ds(j * 16, 16)] = jnp.ones((16,), jnp.float32)

    cp1.wait()

    @pl.loop(0, NCHUNK_CNT)
    def _dc(k):
        for j in range(CH // 16):
            sl = pl.ds(j * 16, 16)
            seg2d[k, sl] = et_all[pl.ds(k * CH + j * 16, 16)]

    cp2 = pltpu.async_copy(et_hbm.at[pl.ds(sid * EPS, EPS)], et_all, sem_in)
    cp2.wait()

    @pl.loop(0, NCHUNK_CNT)
    def _sg(k):
        for j in range(CH // 16):
            sl = pl.ds(j * 16, 16)
            seg2d[k, sl] = seg2d[k, sl] * R + et_all[pl.ds(k * CH + j * 16, 16)]

    plsc.subcore_barrier()

    @pl.loop(0, NCHUNK_CNT)
    def _cnt(k):
        pltpu.async_copy(onesb, cnt_sh.at[seg2d.at[k]], sem_cnt, add=True)

        @pl.when(k >= 4)
        def _w():
            pltpu.make_async_copy(onesb, cnt_sh.at[seg2d.at[k - 4]],
                                  sem_cnt).wait()

    @pl.loop(NCHUNK_CNT - 4, NCHUNK_CNT)
    def _drain(k):
        pltpu.make_async_copy(onesb, cnt_sh.at[seg2d.at[k]], sem_cnt).wait()

    plsc.subcore_barrier()

    pltpu.sync_copy(cnt_sh, cntloc)
    cp3.wait()

    @pl.loop(0, NCHUNK)
    def _sc(k):
        row = cid * NCHUNK + k
        for j in range(CH // 16):
            sl = pl.ds(k * CH + j * 16, 16)
            seg16 = seg2d[row, pl.ds(j * 16, 16)]
            c16 = plsc.load_gather(cntloc, [seg16])
            scb[sl] = 1.0 / jnp.maximum(c16, 1.0)
            d16 = seg16 // R
            e16 = seg16 - d16 * R
            ridx16 = e16 * N + src_half[sl]
            pcb[sl] = lax.shift_left(ridx16, 14) | d16

    pltpu.sync_copy(scb, scale_hbm.at[pl.ds(sid * EPS + cid * EPW, EPW)])
    pltpu.sync_copy(pcb, packed_hbm.at[pl.ds(sid * EPS + cid * EPW, EPW)])


def _edge_scales(edge_index, et):
    mesh = plsc.VectorSubcoreMesh(core_axis_name="c", subcore_axis_name="s")
    return pl.kernel(
        _scale_body,
        out_type=(jax.ShapeDtypeStruct((E,), jnp.float32),
                  jax.ShapeDtypeStruct((E,), jnp.int32)),
        mesh=mesh,
        compiler_params=pltpu.CompilerParams(needs_layout_passes=False),
        scratch_types=[
            pltpu.VMEM_SHARED((NRP,), jnp.float32),
            pltpu.VMEM((NRP,), jnp.float32),
            pltpu.VMEM((NCHUNK_CNT, CH), jnp.int32),
            pltpu.VMEM((EPS,), jnp.int32),
            pltpu.VMEM((EPW,), jnp.int32),
            pltpu.VMEM((EPW,), jnp.float32),
            pltpu.VMEM((EPW,), jnp.int32),
            pltpu.VMEM((CH,), jnp.float32),
            pltpu.VMEM((NRP // NS,), jnp.float32),
            pltpu.SemaphoreType.DMA,
            pltpu.SemaphoreType.DMA,
            pltpu.SemaphoreType.DMA,
        ],
    )(edge_index.reshape(2 * E), et)


def _agg_body(packed_hbm, scale_hbm, xw_hbm, accp_hbm,
              acc_sh, packed_all, sc_all, idxring,
              g0, g1,
              sem_in, sem_g0, sem_g1, sem_s0, sem_s1):
    cid = lax.axis_index("c")
    sid = lax.axis_index("s")
    wid = sid * NC + cid
    ebase = wid * EPW

    cps = [
        pltpu.async_copy(packed_hbm.at[pl.ds(ebase, EPW)], packed_all, sem_in),
        pltpu.async_copy(scale_hbm.at[pl.ds(ebase, EPW)], sc_all, sem_in),
    ]

    @pl.loop(0, 8)
    def _z(i):
        for j in range(H // 16):
            g0[i, pl.ds(j * 16, 16)] = jnp.zeros((16,), jnp.float32)

    @pl.loop(0, RPT // 8)
    def _zc(i):
        pltpu.sync_copy(g0.at[pl.ds(0, 8)],
                        acc_sh.at[pl.ds(sid * RPT + i * 8, 8)])

    @pl.when(sid == NS - 1)
    def _ztail():
        pltpu.sync_copy(g0.at[pl.ds(0, 8)], acc_sh.at[pl.ds(NS * RPT, 8)])
        pltpu.sync_copy(g0.at[pl.ds(0, 8)], acc_sh.at[pl.ds(NS * RPT + 8, 8)])

    for cp in cps:
        cp.wait()

    plsc.subcore_barrier()

    def unpack(k, m):
        for j in range(CH // 16):
            sl = pl.ds(j * 16, 16)
            p16 = packed_all[pl.ds(k * CH + j * 16, 16)]
            idxring[m, sl] = lax.shift_right_logical(p16, 14)
            idxring[2 + m, sl] = lax.bitwise_and(p16, 16383)

    def gather(k, m, gbuf, gsem):
        pltpu.async_copy(xw_hbm.at[idxring.at[m]], gbuf, gsem)

    def wait_gather(k, m, gbuf, gsem):
        pltpu.make_async_copy(xw_hbm.at[idxring.at[m]], gbuf, gsem).wait()

    def scatter(k, m, obuf, ssem):
        pltpu.async_copy(obuf, acc_sh.at[idxring.at[2 + m]], ssem, add=True)

    def wait_scatter(k, m, obuf, ssem):
        pltpu.make_async_copy(obuf, acc_sh.at[idxring.at[2 + m]], ssem).wait()

    splat_dnums = lax.GatherDimensionNumbers(
        offset_dims=(), collapsed_slice_dims=(0,), start_index_map=(0,))

    def scale_mul(k, gbuf):
        for i2 in range(CH // 16):
            base = i2 * 16
            sc16 = sc_all[pl.ds(k * CH + base, 16)]
            for e in range(16):
                sp = lax.gather(
                    sc16, jnp.full((16, 1), e, jnp.int32),
                    dimension_numbers=splat_dnums, slice_sizes=(1,),
                    mode=lax.GatherScatterMode.PROMISE_IN_BOUNDS)
                row = base + e
                for j in range(H // 16):
                    sl = pl.ds(j * 16, 16)
                    gbuf[row, sl] = gbuf[row, sl] * sp

    bufs = ((g0, sem_g0, sem_s0), (g1, sem_g1, sem_s1))

    def chunk_body(k, b, issue_next):
        gb, gs, ss = bufs[b]
        nb = (b + 1) % 2
        ngb, ngs, nss = bufs[nb]
        wait_gather(k, b, gb, gs)
        if issue_next:
            @pl.when(k >= 1)
            def _nx():
                wait_scatter(k - 1, nb, ngb, nss)
                unpack(k + 1, nb)
                gather(k + 1, nb, ngb, ngs)

            @pl.when(k < 1)
            def _nx0():
                unpack(k + 1, nb)
                gather(k + 1, nb, ngb, ngs)

        scale_mul(k, gb)
        scatter(k, b, gb, ss)

    unpack(0, 0)
    gather(0, 0, g0, sem_g0)

    @pl.loop(0, NCHUNK // 2)
    def _run(g):
        for b in range(2):
            chunk_body(2 * g + b, b, True)

    kt = NCHUNK - NCHUNK % 2
    for k in range(kt, NCHUNK):
        chunk_body(jnp.int32(k), k % 2, k + 1 < NCHUNK)

    for k in range(NCHUNK - 2, NCHUNK):
        gb, gs, ss = bufs[k % 2]
        wait_scatter(k, k % 2, gb, ss)

    plsc.subcore_barrier()

    pltpu.sync_copy(acc_sh.at[pl.ds(sid * RPT, RPT)],
                    accp_hbm.at[cid, pl.ds(sid * RPT, RPT)])

    @pl.when(sid == NS - 1)
    def _wtail():
        pltpu.sync_copy(acc_sh.at[pl.ds(NS * RPT, 16)],
                        accp_hbm.at[cid, pl.ds(NS * RPT, 16)])


def _aggregate(packed, scale, xw):
    mesh = plsc.VectorSubcoreMesh(core_axis_name="c", subcore_axis_name="s")
    return pl.kernel(
        _agg_body,
        out_type=jax.ShapeDtypeStruct((NC, N, H), jnp.float32),
        mesh=mesh,
        compiler_params=pltpu.CompilerParams(needs_layout_passes=False),
        scratch_types=[
            pltpu.VMEM_SHARED((N, H), jnp.float32),
            pltpu.VMEM((EPW,), jnp.int32),
            pltpu.VMEM((EPW,), jnp.float32),
            pltpu.VMEM((4, CH), jnp.int32),
            pltpu.VMEM((CH, H), jnp.float32),
            pltpu.VMEM((CH, H), jnp.float32),
            pltpu.SemaphoreType.DMA,
            pltpu.SemaphoreType.DMA,
            pltpu.SemaphoreType.DMA,
            pltpu.SemaphoreType.DMA,
            pltpu.SemaphoreType.DMA,
        ],
    )(packed, scale, xw)


def kernel(x, edge_index, edge_type, W1, root1, b1, W2, root2, b2, fc_w, fc_b):
    w1rel = W1
    w2rel = W2
    b1r = b1.reshape(1, H)
    b2r = b2.reshape(1, H)

    scale, packed = _edge_scales(edge_index, edge_type)

    xwb1, xroot1 = _matmul_rel(x, w1rel, root1, b1r)
    acc1 = _aggregate(packed, scale, xwb1.reshape(R * N, H))

    xwb2, xroot2 = _matmul_rel_fused(acc1, xroot1, w2rel, root2, b2r)
    acc2 = _aggregate(packed, scale, xwb2.reshape(R * N, H))

    return _head(acc2, xroot2, fc_w.reshape(1, H), fc_b.reshape(1, 1))


# speedup vs baseline: 1.2631x; 1.2631x over previous
import functools

import jax
import jax.numpy as jnp
import numpy as np
from jax import lax
from jax.experimental import pallas as pl
from jax.experimental.pallas import tpu as pltpu
from jax.experimental.pallas import tpu_sc as plsc

N = 10000
E = 320000
F = 128
H = 128
R = 3

NC = 2
NS = 16
NW = NC * NS

EPW = E // NW
CH = 80
NCHUNK = EPW // CH
EPS = E // NS
NCHUNK_CNT = EPS // CH
NRP = 30720
RPT = 624

BN = 1000
NB = N // BN


def _mm_body(x_ref, wrel_ref, wroot_ref, b_ref, o_ref, oroot_ref):
    xb = x_ref[...]
    for r in range(R):
        o_ref[r] = jnp.dot(xb, wrel_ref[r], preferred_element_type=jnp.float32)
    oroot_ref[...] = (
        jnp.dot(xb, wroot_ref[...], preferred_element_type=jnp.float32)
        + b_ref[...])


def _matmul_rel(x, wrel, wroot, b):
    return pl.pallas_call(
        _mm_body,
        grid=(NB,),
        in_specs=[
            pl.BlockSpec((BN, F), lambda nb: (nb, 0)),
            pl.BlockSpec((R, F, H), lambda nb: (0, 0, 0)),
            pl.BlockSpec((F, H), lambda nb: (0, 0)),
            pl.BlockSpec((1, H), lambda nb: (0, 0)),
        ],
        out_specs=[
            pl.BlockSpec((R, BN, H), lambda nb: (0, nb, 0)),
            pl.BlockSpec((BN, H), lambda nb: (nb, 0)),
        ],
        out_shape=[
            jax.ShapeDtypeStruct((R, N, H), jnp.float32),
            jax.ShapeDtypeStruct((N, H), jnp.float32),
        ],
    )(x, wrel, wroot, b)


def _mm2_body(acc_ref, xroot_ref, wrel_ref, wroot_ref, b_ref, o_ref,
              oroot_ref):
    h = jnp.maximum(acc_ref[0] + acc_ref[1] + xroot_ref[...], 0.0)
    for r in range(R):
        o_ref[r] = jnp.dot(h, wrel_ref[r], preferred_element_type=jnp.float32)
    oroot_ref[...] = (
        jnp.dot(h, wroot_ref[...], preferred_element_type=jnp.float32)
        + b_ref[...])


def _matmul_rel_fused(accp, xroot, wrel, wroot, b):
    return pl.pallas_call(
        _mm2_body,
        grid=(NB,),
        in_specs=[
            pl.BlockSpec((2, BN, H), lambda nb: (0, nb, 0)),
            pl.BlockSpec((BN, H), lambda nb: (nb, 0)),
            pl.BlockSpec((R, H, H), lambda nb: (0, 0, 0)),
            pl.BlockSpec((H, H), lambda nb: (0, 0)),
            pl.BlockSpec((1, H), lambda nb: (0, 0)),
        ],
        out_specs=[
            pl.BlockSpec((R, BN, H), lambda nb: (0, nb, 0)),
            pl.BlockSpec((BN, H), lambda nb: (nb, 0)),
        ],
        out_shape=[
            jax.ShapeDtypeStruct((R, N, H), jnp.float32),
            jax.ShapeDtypeStruct((N, H), jnp.float32),
        ],
    )(accp, xroot, wrel, wroot, b)


def _head_body(acc_ref, xroot_ref, fcw_ref, fcb_ref, o_ref):
    h = jnp.maximum(acc_ref[0] + acc_ref[1] + xroot_ref[...], 0.0)
    z = jnp.sum(h * fcw_ref[...], axis=1, keepdims=True) + fcb_ref[...]
    o_ref[...] = 1.0 / (1.0 + jnp.exp(-z))


def _head(accp, xroot, fcw_row, fcb):
    return pl.pallas_call(
        _head_body,
        grid=(NB,),
        in_specs=[
            pl.BlockSpec((2, BN, H), lambda nb: (0, nb, 0)),
            pl.BlockSpec((BN, H), lambda nb: (nb, 0)),
            pl.BlockSpec((1, H), lambda nb: (0, 0)),
            pl.BlockSpec((1, 1), lambda nb: (0, 0)),
        ],
        out_specs=pl.BlockSpec((BN, 1), lambda nb: (nb, 0)),
        out_shape=jax.ShapeDtypeStruct((N, 1), jnp.float32),
    )(accp, xroot, fcw_row, fcb)


def _scale_body(ei_hbm, et_hbm, scale_hbm, packed_hbm,
                cnt_sh, cntloc, seg2d, et_all, src_half, scb, pcb, onesb, zb,
                sem_in, sem_cnt, sem_src):
    cid = lax.axis_index("c")
    sid = lax.axis_index("s")

    cp1 = pltpu.async_copy(ei_hbm.at[pl.ds(E + sid * EPS, EPS)], et_all,
                           sem_in)
    cp3 = pltpu.async_copy(ei_hbm.at[pl.ds(sid * EPS + cid * EPW, EPW)],
                           src_half, sem_src)

    @pl.loop(0, NRP // NS // 16)
    def _z(i):
        zb[pl.ds(i * 16, 16)] = jnp.zeros((16,), jnp.float32)

    pltpu.sync_copy(zb, cnt_sh.at[pl.ds(sid * (NRP // NS), NRP // NS)])

    @pl.loop(0, CH // 16)
    def _o(j):
        onesb[pl.ds(j * 16, 16)] = jnp.ones((16,), jnp.float32)

    cp1.wait()

    @pl.loop(0, NCHUNK_CNT)
    def _dc(k):
        for j in range(CH // 16):
            sl = pl.ds(j * 16, 16)
            seg2d[k, sl] = et_all[pl.ds(k * CH + j * 16, 16)]

    cp2 = pltpu.async_copy(et_hbm.at[pl.ds(sid * EPS, EPS)], et_all, sem_in)
    cp2.wait()

    @pl.loop(0, NCHUNK_CNT)
    def _sg(k):
        for j in range(CH // 16):
            sl = pl.ds(j * 16, 16)
            seg2d[k, sl] = seg2d[k, sl] * R + et_all[pl.ds(k * CH + j * 16, 16)]

    plsc.subcore_barrier()

    @pl.loop(0, NCHUNK_CNT)
    def _cnt(k):
        pltpu.async_copy(onesb, cnt_sh.at[seg2d.at[k]], sem_cnt, add=True)

        @pl.when(k >= 4)
        def _w():
            pltpu.make_async_copy(onesb, cnt_sh.at[seg2d.at[k - 4]],
                                  sem_cnt).wait()

    @pl.loop(NCHUNK_CNT - 4, NCHUNK_CNT)
    def _drain(k):
        pltpu.make_async_copy(onesb, cnt_sh.at[seg2d.at[k]], sem_cnt).wait()

    plsc.subcore_barrier()

    pltpu.sync_copy(cnt_sh, cntloc)
    cp3.wait()

    @pl.loop(0, NCHUNK)
    def _sc(k):
        row = cid * NCHUNK + k
        for j in range(CH // 16):
            sl = pl.ds(k * CH + j * 16, 16)
            seg16 = seg2d[row, pl.ds(j * 16, 16)]
            c16 = plsc.load_gather(cntloc, [seg16])
            scb[sl] = 1.0 / jnp.maximum(c16, 1.0)
            d16 = seg16 // R
            e16 = seg16 - d16 * R
            ridx16 = e16 * N + src_half[sl]
            pcb[sl] = lax.shift_left(ridx16, 14) | d16

    pltpu.sync_copy(scb, scale_hbm.at[pl.ds(sid * EPS + cid * EPW, EPW)])
    pltpu.sync_copy(pcb, packed_hbm.at[pl.ds(sid * EPS + cid * EPW, EPW)])


def _edge_scales(edge_index, et):
    mesh = plsc.VectorSubcoreMesh(core_axis_name="c", subcore_axis_name="s")
    return pl.kernel(
        _scale_body,
        out_type=(jax.ShapeDtypeStruct((E,), jnp.float32),
                  jax.ShapeDtypeStruct((E,), jnp.int32)),
        mesh=mesh,
        compiler_params=pltpu.CompilerParams(needs_layout_passes=False),
        scratch_types=[
            pltpu.VMEM_SHARED((NRP,), jnp.float32),
            pltpu.VMEM((NRP,), jnp.float32),
            pltpu.VMEM((NCHUNK_CNT, CH), jnp.int32),
            pltpu.VMEM((EPS,), jnp.int32),
            pltpu.VMEM((EPW,), jnp.int32),
            pltpu.VMEM((EPW,), jnp.float32),
            pltpu.VMEM((EPW,), jnp.int32),
            pltpu.VMEM((CH,), jnp.float32),
            pltpu.VMEM((NRP // NS,), jnp.float32),
            pltpu.SemaphoreType.DMA,
            pltpu.SemaphoreType.DMA,
            pltpu.SemaphoreType.DMA,
        ],
    )(edge_index.reshape(2 * E), et)


def _agg_body(packed_hbm, scale_hbm, xw_hbm, accp_hbm,
              acc_sh, packed_all, sc_all, idxring,
              g0, g1,
              sem_in, sem_g0, sem_g1, sem_s0, sem_s1):
    cid = lax.axis_index("c")
    sid = lax.axis_index("s")
    wid = sid * NC + cid
    ebase = wid * EPW

    cps = [
        pltpu.async_copy(packed_hbm.at[pl.ds(ebase, EPW)], packed_all, sem_in),
        pltpu.async_copy(scale_hbm.at[pl.ds(ebase, EPW)], sc_all, sem_in),
    ]

    @pl.loop(0, 8)
    def _z(i):
        for j in range(H // 16):
            g0[i, pl.ds(j * 16, 16)] = jnp.zeros((16,), jnp.float32)

    @pl.loop(0, RPT // 8)
    def _zc(i):
        pltpu.sync_copy(g0.at[pl.ds(0, 8)],
                        acc_sh.at[pl.ds(sid * RPT + i * 8, 8)])

    @pl.when(sid == NS - 1)
    def _ztail():
        pltpu.sync_copy(g0.at[pl.ds(0, 8)], acc_sh.at[pl.ds(NS * RPT, 8)])
        pltpu.sync_copy(g0.at[pl.ds(0, 8)], acc_sh.at[pl.ds(NS * RPT + 8, 8)])

    for cp in cps:
        cp.wait()

    plsc.subcore_barrier()

    def unpack(k, m):
        for j in range(CH // 16):
            sl = pl.ds(j * 16, 16)
            p16 = packed_all[pl.ds(k * CH + j * 16, 16)]
            idxring[m, sl] = lax.shift_right_logical(p16, 14)
            idxring[2 + m, sl] = lax.bitwise_and(p16, 16383)

    def gather(k, m, gbuf, gsem):
        pltpu.async_copy(xw_hbm.at[idxring.at[m]], gbuf, gsem)

    def wait_gather(k, m, gbuf, gsem):
        pltpu.make_async_copy(xw_hbm.at[idxring.at[m]], gbuf, gsem).wait()

    def scatter(k, m, obuf, ssem):
        pltpu.async_copy(obuf, acc_sh.at[idxring.at[2 + m]], ssem, add=True)

    def wait_scatter(k, m, obuf, ssem):
        pltpu.make_async_copy(obuf, acc_sh.at[idxring.at[2 + m]], ssem).wait()

    splat_dnums = lax.GatherDimensionNumbers(
        offset_dims=(), collapsed_slice_dims=(0,), start_index_map=(0,))

    def scale_mul(k, gbuf):
        @pl.loop(0, CH // 16)
        def _m16(i2):
            base = i2 * 16
            sc16 = sc_all[pl.ds(k * CH + base, 16)]
            for e in range(16):
                sp = lax.gather(
                    sc16, jnp.full((16, 1), e, jnp.int32),
                    dimension_numbers=splat_dnums, slice_sizes=(1,),
                    mode=lax.GatherScatterMode.PROMISE_IN_BOUNDS)
                row = base + e
                for j in range(H // 16):
                    sl = pl.ds(j * 16, 16)
                    gbuf[row, sl] = gbuf[row, sl] * sp

    bufs = ((g0, sem_g0, sem_s0), (g1, sem_g1, sem_s1))

    def chunk_body(k, b, issue_next):
        gb, gs, ss = bufs[b]
        nb = (b + 1) % 2
        ngb, ngs, nss = bufs[nb]
        if issue_next:
            @pl.when(k >= 1)
            def _nx():
                wait_scatter(k - 1, nb, ngb, nss)
                unpack(k + 1, nb)
                gather(k + 1, nb, ngb, ngs)

            @pl.when(k < 1)
            def _nx0():
                unpack(k + 1, nb)
                gather(k + 1, nb, ngb, ngs)

        wait_gather(k, b, gb, gs)
        scale_mul(k, gb)
        scatter(k, b, gb, ss)

    unpack(0, 0)
    gather(0, 0, g0, sem_g0)

    @pl.loop(0, NCHUNK // 2)
    def _run(g):
        for b in range(2):
            chunk_body(2 * g + b, b, True)

    kt = NCHUNK - NCHUNK % 2
    for k in range(kt, NCHUNK):
        chunk_body(jnp.int32(k), k % 2, k + 1 < NCHUNK)

    for k in range(NCHUNK - 2, NCHUNK):
        gb, gs, ss = bufs[k % 2]
        wait_scatter(k, k % 2, gb, ss)

    plsc.subcore_barrier()

    pltpu.sync_copy(acc_sh.at[pl.ds(sid * RPT, RPT)],
                    accp_hbm.at[cid, pl.ds(sid * RPT, RPT)])

    @pl.when(sid == NS - 1)
    def _wtail():
        pltpu.sync_copy(acc_sh.at[pl.ds(NS * RPT, 16)],
                        accp_hbm.at[cid, pl.ds(NS * RPT, 16)])


def _aggregate(packed, scale, xw):
    mesh = plsc.VectorSubcoreMesh(core_axis_name="c", subcore_axis_name="s")
    return pl.kernel(
        _agg_body,
        out_type=jax.ShapeDtypeStruct((NC, N, H), jnp.float32),
        mesh=mesh,
        compiler_params=pltpu.CompilerParams(needs_layout_passes=False),
        scratch_types=[
            pltpu.VMEM_SHARED((N, H), jnp.float32),
            pltpu.VMEM((EPW,), jnp.int32),
            pltpu.VMEM((EPW,), jnp.float32),
            pltpu.VMEM((4, CH), jnp.int32),
            pltpu.VMEM((CH, H), jnp.float32),
            pltpu.VMEM((CH, H), jnp.float32),
            pltpu.SemaphoreType.DMA,
            pltpu.SemaphoreType.DMA,
            pltpu.SemaphoreType.DMA,
            pltpu.SemaphoreType.DMA,
            pltpu.SemaphoreType.DMA,
        ],
    )(packed, scale, xw)


def kernel(x, edge_index, edge_type, W1, root1, b1, W2, root2, b2, fc_w, fc_b):
    w1rel = W1
    w2rel = W2
    b1r = b1.reshape(1, H)
    b2r = b2.reshape(1, H)

    scale, packed = _edge_scales(edge_index, edge_type)

    xwb1, xroot1 = _matmul_rel(x, w1rel, root1, b1r)
    acc1 = _aggregate(packed, scale, xwb1.reshape(R * N, H))

    xwb2, xroot2 = _matmul_rel_fused(acc1, xroot1, w2rel, root2, b2r)
    acc2 = _aggregate(packed, scale, xwb2.reshape(R * N, H))

    return _head(acc2, xroot2, fc_w.reshape(1, H), fc_b.reshape(1, 1))


# speedup vs baseline: 1.2739x; 1.0086x over previous
import functools

import jax
import jax.numpy as jnp
import numpy as np
from jax import lax
from jax.experimental import pallas as pl
from jax.experimental.pallas import tpu as pltpu
from jax.experimental.pallas import tpu_sc as plsc

N = 10000
E = 320000
F = 128
H = 128
R = 3

NC = 2
NS = 16
NW = NC * NS

EPW = E // NW
CH = 80
NCHUNK = EPW // CH
EPS = E // NS
NCHUNK_CNT = EPS // CH
NRP = 30720
RPT = 624

BN = 1000
NB = N // BN


def _mm_body(x_ref, wrel_ref, wroot_ref, b_ref, o_ref, oroot_ref):
    xb = x_ref[...]
    for r in range(R):
        o_ref[r] = jnp.dot(xb, wrel_ref[r], preferred_element_type=jnp.float32)
    oroot_ref[...] = (
        jnp.dot(xb, wroot_ref[...], preferred_element_type=jnp.float32)
        + b_ref[...])


def _matmul_rel(x, wrel, wroot, b):
    return pl.pallas_call(
        _mm_body,
        grid=(NB,),
        in_specs=[
            pl.BlockSpec((BN, F), lambda nb: (nb, 0)),
            pl.BlockSpec((R, F, H), lambda nb: (0, 0, 0)),
            pl.BlockSpec((F, H), lambda nb: (0, 0)),
            pl.BlockSpec((1, H), lambda nb: (0, 0)),
        ],
        out_specs=[
            pl.BlockSpec((R, BN, H), lambda nb: (0, nb, 0)),
            pl.BlockSpec((BN, H), lambda nb: (nb, 0)),
        ],
        out_shape=[
            jax.ShapeDtypeStruct((R, N, H), jnp.float32),
            jax.ShapeDtypeStruct((N, H), jnp.float32),
        ],
    )(x, wrel, wroot, b)


def _mm2_body(acc_ref, xroot_ref, wrel_ref, wroot_ref, b_ref, o_ref,
              oroot_ref):
    h = jnp.maximum(acc_ref[0] + acc_ref[1] + xroot_ref[...], 0.0)
    for r in range(R):
        o_ref[r] = jnp.dot(h, wrel_ref[r], preferred_element_type=jnp.float32)
    oroot_ref[...] = (
        jnp.dot(h, wroot_ref[...], preferred_element_type=jnp.float32)
        + b_ref[...])


def _matmul_rel_fused(accp, xroot, wrel, wroot, b):
    return pl.pallas_call(
        _mm2_body,
        grid=(NB,),
        in_specs=[
            pl.BlockSpec((2, BN, H), lambda nb: (0, nb, 0)),
            pl.BlockSpec((BN, H), lambda nb: (nb, 0)),
            pl.BlockSpec((R, H, H), lambda nb: (0, 0, 0)),
            pl.BlockSpec((H, H), lambda nb: (0, 0)),
            pl.BlockSpec((1, H), lambda nb: (0, 0)),
        ],
        out_specs=[
            pl.BlockSpec((R, BN, H), lambda nb: (0, nb, 0)),
            pl.BlockSpec((BN, H), lambda nb: (nb, 0)),
        ],
        out_shape=[
            jax.ShapeDtypeStruct((R, N, H), jnp.float32),
            jax.ShapeDtypeStruct((N, H), jnp.float32),
        ],
    )(accp, xroot, wrel, wroot, b)


def _head_body(acc_ref, xroot_ref, fcw_ref, fcb_ref, o_ref):
    h = jnp.maximum(acc_ref[0] + acc_ref[1] + xroot_ref[...], 0.0)
    z = jnp.sum(h * fcw_ref[...], axis=1, keepdims=True) + fcb_ref[...]
    o_ref[...] = 1.0 / (1.0 + jnp.exp(-z))


def _head(accp, xroot, fcw_row, fcb):
    return pl.pallas_call(
        _head_body,
        grid=(NB,),
        in_specs=[
            pl.BlockSpec((2, BN, H), lambda nb: (0, nb, 0)),
            pl.BlockSpec((BN, H), lambda nb: (nb, 0)),
            pl.BlockSpec((1, H), lambda nb: (0, 0)),
            pl.BlockSpec((1, 1), lambda nb: (0, 0)),
        ],
        out_specs=pl.BlockSpec((BN, 1), lambda nb: (nb, 0)),
        out_shape=jax.ShapeDtypeStruct((N, 1), jnp.float32),
    )(accp, xroot, fcw_row, fcb)


def _scale_body(ei_hbm, et_hbm, scale_hbm, packed_hbm,
                cnt_sh, cntloc, seg2d, et_all, src_half, scb, pcb, onesb, zb,
                sem_in, sem_cnt, sem_src):
    cid = lax.axis_index("c")
    sid = lax.axis_index("s")

    cp1 = pltpu.async_copy(ei_hbm.at[pl.ds(E + sid * EPS, EPS)], et_all,
                           sem_in)
    cp3 = pltpu.async_copy(ei_hbm.at[pl.ds(sid * EPS + cid * EPW, EPW)],
                           src_half, sem_src)

    @pl.loop(0, NRP // NS // 16)
    def _z(i):
        zb[pl.ds(i * 16, 16)] = jnp.zeros((16,), jnp.float32)

    pltpu.sync_copy(zb, cnt_sh.at[pl.ds(sid * (NRP // NS), NRP // NS)])

    @pl.loop(0, CH // 16)
    def _o(j):
        onesb[pl.ds(j * 16, 16)] = jnp.ones((16,), jnp.float32)

    cp1.wait()

    @pl.loop(0, NCHUNK_CNT)
    def _dc(k):
        for j in range(CH // 16):
            sl = pl.ds(j * 16, 16)
            seg2d[k, sl] = et_all[pl.ds(k * CH + j * 16, 16)]

    cp2 = pltpu.async_copy(et_hbm.at[pl.ds(sid * EPS, EPS)], et_all, sem_in)
    cp2.wait()

    @pl.loop(0, NCHUNK_CNT)
    def _sg(k):
        for j in range(CH // 16):
            sl = pl.ds(j * 16, 16)
            seg2d[k, sl] = seg2d[k, sl] * R + et_all[pl.ds(k * CH + j * 16, 16)]

    plsc.subcore_barrier()

    @pl.loop(0, NCHUNK_CNT)
    def _cnt(k):
        pltpu.async_copy(onesb, cnt_sh.at[seg2d.at[k]], sem_cnt, add=True)

        @pl.when(k >= 12)
        def _w():
            pltpu.make_async_copy(onesb, cnt_sh.at[seg2d.at[k - 12]],
                                  sem_cnt).wait()

    @pl.loop(NCHUNK_CNT - 12, NCHUNK_CNT)
    def _drain(k):
        pltpu.make_async_copy(onesb, cnt_sh.at[seg2d.at[k]], sem_cnt).wait()

    plsc.subcore_barrier()

    pltpu.sync_copy(cnt_sh, cntloc)
    cp3.wait()

    @pl.loop(0, NCHUNK)
    def _sc(k):
        row = cid * NCHUNK + k
        for j in range(CH // 16):
            sl = pl.ds(k * CH + j * 16, 16)
            seg16 = seg2d[row, pl.ds(j * 16, 16)]
            c16 = plsc.load_gather(cntloc, [seg16])
            scb[sl] = 1.0 / jnp.maximum(c16, 1.0)
            d16 = seg16 // R
            e16 = seg16 - d16 * R
            ridx16 = e16 * N + src_half[sl]
            pcb[sl] = lax.shift_left(ridx16, 14) | d16

    pltpu.sync_copy(scb, scale_hbm.at[pl.ds(sid * EPS + cid * EPW, EPW)])
    pltpu.sync_copy(pcb, packed_hbm.at[pl.ds(sid * EPS + cid * EPW, EPW)])


def _edge_scales(edge_index, et):
    mesh = plsc.VectorSubcoreMesh(core_axis_name="c", subcore_axis_name="s")
    return pl.kernel(
        _scale_body,
        out_type=(jax.ShapeDtypeStruct((E,), jnp.float32),
                  jax.ShapeDtypeStruct((E,), jnp.int32)),
        mesh=mesh,
        compiler_params=pltpu.CompilerParams(needs_layout_passes=False),
        scratch_types=[
            pltpu.VMEM_SHARED((NRP,), jnp.float32),
            pltpu.VMEM((NRP,), jnp.float32),
            pltpu.VMEM((NCHUNK_CNT, CH), jnp.int32),
            pltpu.VMEM((EPS,), jnp.int32),
            pltpu.VMEM((EPW,), jnp.int32),
            pltpu.VMEM((EPW,), jnp.float32),
            pltpu.VMEM((EPW,), jnp.int32),
            pltpu.VMEM((CH,), jnp.float32),
            pltpu.VMEM((NRP // NS,), jnp.float32),
            pltpu.SemaphoreType.DMA,
            pltpu.SemaphoreType.DMA,
            pltpu.SemaphoreType.DMA,
        ],
    )(edge_index.reshape(2 * E), et)


def _agg_body(packed_hbm, scale_hbm, xw_hbm, accp_hbm,
              acc_sh, packed_all, sc_all, idxring,
              g0, g1,
              sem_in, sem_g0, sem_g1, sem_s0, sem_s1):
    cid = lax.axis_index("c")
    sid = lax.axis_index("s")
    wid = sid * NC + cid
    ebase = wid * EPW

    cps = [
        pltpu.async_copy(packed_hbm.at[pl.ds(ebase, EPW)], packed_all, sem_in),
        pltpu.async_copy(scale_hbm.at[pl.ds(ebase, EPW)], sc_all, sem_in),
    ]

    @pl.loop(0, 8)
    def _z(i):
        for j in range(H // 16):
            g0[i, pl.ds(j * 16, 16)] = jnp.zeros((16,), jnp.float32)

    @pl.loop(0, RPT // 8)
    def _zc(i):
        pltpu.sync_copy(g0.at[pl.ds(0, 8)],
                        acc_sh.at[pl.ds(sid * RPT + i * 8, 8)])

    @pl.when(sid == NS - 1)
    def _ztail():
        pltpu.sync_copy(g0.at[pl.ds(0, 8)], acc_sh.at[pl.ds(NS * RPT, 8)])
        pltpu.sync_copy(g0.at[pl.ds(0, 8)], acc_sh.at[pl.ds(NS * RPT + 8, 8)])

    for cp in cps:
        cp.wait()

    plsc.subcore_barrier()

    def unpack(k, m):
        for j in range(CH // 16):
            sl = pl.ds(j * 16, 16)
            p16 = packed_all[pl.ds(k * CH + j * 16, 16)]
            idxring[m, sl] = lax.shift_right_logical(p16, 14)
            idxring[2 + m, sl] = lax.bitwise_and(p16, 16383)

    def gather(k, m, gbuf, gsem):
        pltpu.async_copy(xw_hbm.at[idxring.at[m]], gbuf, gsem)

    def wait_gather(k, m, gbuf, gsem):
        pltpu.make_async_copy(xw_hbm.at[idxring.at[m]], gbuf, gsem).wait()

    def scatter(k, m, obuf, ssem):
        pltpu.async_copy(obuf, acc_sh.at[idxring.at[2 + m]], ssem, add=True)

    def wait_scatter(k, m, obuf, ssem):
        pltpu.make_async_copy(obuf, acc_sh.at[idxring.at[2 + m]], ssem).wait()

    splat_dnums = lax.GatherDimensionNumbers(
        offset_dims=(), collapsed_slice_dims=(0,), start_index_map=(0,))

    def scale_mul(k, gbuf):
        @pl.loop(0, CH // 16)
        def _m16(i2):
            base = i2 * 16
            sc16 = sc_all[pl.ds(k * CH + base, 16)]
            for e in range(16):
                sp = lax.gather(
                    sc16, jnp.full((16, 1), e, jnp.int32),
                    dimension_numbers=splat_dnums, slice_sizes=(1,),
                    mode=lax.GatherScatterMode.PROMISE_IN_BOUNDS)
                row = base + e
                for j in range(H // 16):
                    sl = pl.ds(j * 16, 16)
                    gbuf[row, sl] = gbuf[row, sl] * sp

    bufs = ((g0, sem_g0, sem_s0), (g1, sem_g1, sem_s1))

    def chunk_body(k, b, issue_next):
        gb, gs, ss = bufs[b]
        nb = (b + 1) % 2
        ngb, ngs, nss = bufs[nb]
        if issue_next:
            @pl.when(k >= 1)
            def _nx():
                wait_scatter(k - 1, nb, ngb, nss)
                unpack(k + 1, nb)
                gather(k + 1, nb, ngb, ngs)

            @pl.when(k < 1)
            def _nx0():
                unpack(k + 1, nb)
                gather(k + 1, nb, ngb, ngs)

        wait_gather(k, b, gb, gs)
        scale_mul(k, gb)
        scatter(k, b, gb, ss)

    unpack(0, 0)
    gather(0, 0, g0, sem_g0)

    @pl.loop(0, NCHUNK // 2)
    def _run(g):
        for b in range(2):
            chunk_body(2 * g + b, b, True)

    kt = NCHUNK - NCHUNK % 2
    for k in range(kt, NCHUNK):
        chunk_body(jnp.int32(k), k % 2, k + 1 < NCHUNK)

    for k in range(NCHUNK - 2, NCHUNK):
        gb, gs, ss = bufs[k % 2]
        wait_scatter(k, k % 2, gb, ss)

    plsc.subcore_barrier()

    pltpu.sync_copy(acc_sh.at[pl.ds(sid * RPT, RPT)],
                    accp_hbm.at[cid, pl.ds(sid * RPT, RPT)])

    @pl.when(sid == NS - 1)
    def _wtail():
        pltpu.sync_copy(acc_sh.at[pl.ds(NS * RPT, 16)],
                        accp_hbm.at[cid, pl.ds(NS * RPT, 16)])


def _aggregate(packed, scale, xw):
    mesh = plsc.VectorSubcoreMesh(core_axis_name="c", subcore_axis_name="s")
    return pl.kernel(
        _agg_body,
        out_type=jax.ShapeDtypeStruct((NC, N, H), jnp.float32),
        mesh=mesh,
        compiler_params=pltpu.CompilerParams(needs_layout_passes=False),
        scratch_types=[
            pltpu.VMEM_SHARED((N, H), jnp.float32),
            pltpu.VMEM((EPW,), jnp.int32),
            pltpu.VMEM((EPW,), jnp.float32),
            pltpu.VMEM((4, CH), jnp.int32),
            pltpu.VMEM((CH, H), jnp.float32),
            pltpu.VMEM((CH, H), jnp.float32),
            pltpu.SemaphoreType.DMA,
            pltpu.SemaphoreType.DMA,
            pltpu.SemaphoreType.DMA,
            pltpu.SemaphoreType.DMA,
            pltpu.SemaphoreType.DMA,
        ],
    )(packed, scale, xw)


def kernel(x, edge_index, edge_type, W1, root1, b1, W2, root2, b2, fc_w, fc_b):
    w1rel = W1
    w2rel = W2
    b1r = b1.reshape(1, H)
    b2r = b2.reshape(1, H)

    scale, packed = _edge_scales(edge_index, edge_type)

    xwb1, xroot1 = _matmul_rel(x, w1rel, root1, b1r)
    acc1 = _aggregate(packed, scale, xwb1.reshape(R * N, H))

    xwb2, xroot2 = _matmul_rel_fused(acc1, xroot1, w2rel, root2, b2r)
    acc2 = _aggregate(packed, scale, xwb2.reshape(R * N, H))

    return _head(acc2, xroot2, fc_w.reshape(1, H), fc_b.reshape(1, 1))
